# Initial kernel scaffold; baseline (speedup 1.0000x reference)
import jax
import jax.numpy as jnp
from jax.experimental import pallas as pl


def kernel(hv, edge_index, he, Wm0, bm0, Wih0, Whh0, bih0, bhh0, Wm1, bm1, Wih1, Whh1, bih1, bhh1):
    raise NotImplementedError("write your pallas kernel here")


# speedup vs baseline: 1.3213x; 1.3213x over previous
import functools

import jax
import jax.numpy as jnp
from jax import lax
from jax.experimental import pallas as pl
from jax.experimental.pallas import tpu as pltpu
from jax.experimental.pallas import tpu_sc as plsc

N = 10000
E = 320000
D = 128
DE = 16

NC = 2
NS = 16
NW = NC * NS
CHUNK = 128
CPT = 80
TOTAL_CHUNKS = CPT * NW
EPAD = TOTAL_CHUNKS * CHUNK
N_SC = 10240
RPT = N_SC // NS
RB = RPT // CHUNK
BE = 2048
BN = 512


def _gather_body(hv_hbm, ei_hbm, gs_hbm, gd_hbm, sidx, didx, rows, sem):
    cid = lax.axis_index("c")
    sid = lax.axis_index("s")
    wid = sid * NC + cid

    @pl.loop(0, CPT)
    def _edges(i):
        chunk = wid * CPT + i
        pltpu.sync_copy(ei_hbm.at[chunk, 0], sidx)
        pltpu.sync_copy(ei_hbm.at[chunk, 1], didx)
        pltpu.async_copy(hv_hbm.at[sidx], rows, sem).wait()
        pltpu.sync_copy(rows, gs_hbm.at[chunk])
        pltpu.async_copy(hv_hbm.at[didx], rows, sem).wait()
        pltpu.sync_copy(rows, gd_hbm.at[chunk])


_gather = functools.partial(
    pl.kernel,
    out_type=(jax.ShapeDtypeStruct((TOTAL_CHUNKS, CHUNK, D), jnp.float32),
              jax.ShapeDtypeStruct((TOTAL_CHUNKS, CHUNK, D), jnp.float32)),
    mesh=plsc.VectorSubcoreMesh(core_axis_name="c", subcore_axis_name="s"),
    scratch_types=[
        pltpu.VMEM((CHUNK,), jnp.int32),
        pltpu.VMEM((CHUNK,), jnp.int32),
        pltpu.VMEM((CHUNK, D), jnp.float32),
        pltpu.SemaphoreType.DMA,
    ],
)(_gather_body)


def _scatter_body(act_hbm, ei_hbm, out_hbm, didx, rows, acc, sem):
    cid = lax.axis_index("c")
    sid = lax.axis_index("s")
    wid = sid * NC + cid
    base = sid * RPT

    @pl.loop(0, CHUNK)
    def _zero(i):
        for j in range(D // 16):
            rows[i, pl.ds(j * 16, 16)] = jnp.zeros((16,), jnp.float32)

    for j in range(RB):
        pltpu.sync_copy(rows, acc.at[pl.ds(base + j * CHUNK, CHUNK)])
    plsc.subcore_barrier()

    @pl.loop(0, CPT)
    def _edges(i):
        chunk = wid * CPT + i
        pltpu.sync_copy(ei_hbm.at[chunk, 1], didx)
        pltpu.async_copy(act_hbm.at[chunk], rows, sem).wait()
        pltpu.sync_copy(rows, acc.at[didx], add=True)

    plsc.subcore_barrier()
    for j in range(RB):
        pltpu.sync_copy(acc.at[pl.ds(base + j * CHUNK, CHUNK)], rows)
        pltpu.sync_copy(rows, out_hbm.at[cid, pl.ds(base + j * CHUNK, CHUNK)])


_scatter = functools.partial(
    pl.kernel,
    out_type=jax.ShapeDtypeStruct((NC, N_SC, D), jnp.float32),
    mesh=plsc.VectorSubcoreMesh(core_axis_name="c", subcore_axis_name="s"),
    scratch_types=[
        pltpu.VMEM((CHUNK,), jnp.int32),
        pltpu.VMEM((CHUNK, D), jnp.float32),
        pltpu.VMEM_SHARED((N_SC, D), jnp.float32),
        pltpu.SemaphoreType.DMA,
    ],
)(_scatter_body)


def _act_body(gs, gd, hee, WsT, WdT, WeT, bm_r, out):
    out[...] = (jnp.dot(gs[...], WsT[...], preferred_element_type=jnp.float32)
                + jnp.dot(gd[...], WdT[...], preferred_element_type=jnp.float32)
                + jnp.dot(hee[...], WeT[...], preferred_element_type=jnp.float32)
                + bm_r[...])


def _act(gs, gd, hee, WsT, WdT, WeT, bm_r):
    const = lambda i: (0, 0)
    return pl.pallas_call(
        _act_body,
        grid=(EPAD // BE,),
        in_specs=[
            pl.BlockSpec((BE, D), lambda i: (i, 0)),
            pl.BlockSpec((BE, D), lambda i: (i, 0)),
            pl.BlockSpec((BE, DE), lambda i: (i, 0)),
            pl.BlockSpec((D, D), const),
            pl.BlockSpec((D, D), const),
            pl.BlockSpec((DE, D), const),
            pl.BlockSpec((1, D), const),
        ],
        out_specs=pl.BlockSpec((BE, D), lambda i: (i, 0)),
        out_shape=jax.ShapeDtypeStruct((EPAD, D), jnp.float32),
    )(gs, gd, hee, WsT, WdT, WeT, bm_r)


def _dense_body(ap, hvp, WihT, bih_r, WhhT, bhh_r, out):
    a = ap[0] + ap[1]
    hv = hvp[...]
    gi = jnp.dot(a, WihT[...], preferred_element_type=jnp.float32) + bih_r[...]
    gh = jnp.dot(hv, WhhT[...], preferred_element_type=jnp.float32) + bhh_r[...]
    r = jax.nn.sigmoid(gi[:, :D] + gh[:, :D])
    z = jax.nn.sigmoid(gi[:, D:2 * D] + gh[:, D:2 * D])
    n = jnp.tanh(gi[:, 2 * D:] + r * gh[:, 2 * D:])
    out[...] = (1.0 - z) * n + z * hv


def _dense(ap, hvp, WihT, bih_r, WhhT, bhh_r):
    const = lambda i: (0, 0)
    return pl.pallas_call(
        _dense_body,
        grid=(N_SC // BN,),
        in_specs=[
            pl.BlockSpec((NC, BN, D), lambda i: (0, i, 0)),
            pl.BlockSpec((BN, D), lambda i: (i, 0)),
            pl.BlockSpec((D, 3 * D), const),
            pl.BlockSpec((1, 3 * D), const),
            pl.BlockSpec((D, 3 * D), const),
            pl.BlockSpec((1, 3 * D), const),
        ],
        out_specs=pl.BlockSpec((BN, D), lambda i: (i, 0)),
        out_shape=jax.ShapeDtypeStruct((N_SC, D), jnp.float32),
    )(ap, hvp, WihT, bih_r, WhhT, bhh_r)


def kernel(hv, edge_index, he,
           Wm0, bm0, Wih0, Whh0, bih0, bhh0,
           Wm1, bm1, Wih1, Whh1, bih1, bhh1):
    src = edge_index[0]
    dst = edge_index[1]
    pad_e = EPAD - E
    src_p = jnp.concatenate(
        [src, jnp.zeros((pad_e,), jnp.int32)]).reshape(TOTAL_CHUNKS, CHUNK)
    dst_p = jnp.concatenate(
        [dst, jnp.full((pad_e,), N, jnp.int32)]).reshape(TOTAL_CHUNKS, CHUNK)
    ei_p = jnp.stack([src_p, dst_p], axis=1)
    he_p = jnp.concatenate(
        [he, jnp.zeros((pad_e, DE), jnp.float32)])
    hv_p = jnp.pad(hv, ((0, N_SC - N), (0, 0)))

    for (Wm, bm, Wih, Whh, bih, bhh) in (
            (Wm0, bm0, Wih0, Whh0, bih0, bhh0),
            (Wm1, bm1, Wih1, Whh1, bih1, bhh1)):
        gs, gd = _gather(hv_p, ei_p)
        act = _act(gs.reshape(EPAD, D), gd.reshape(EPAD, D), he_p,
                   Wm[:, :D].T, Wm[:, D:2 * D].T, Wm[:, 2 * D:].T,
                   bm.reshape(1, D))
        ap = _scatter(act.reshape(TOTAL_CHUNKS, CHUNK, D), ei_p)
        hv_p = _dense(ap, hv_p, Wih.T, bih.reshape(1, 3 * D),
                      Whh.T, bhh.reshape(1, 3 * D))
    return hv_p[:N]


# speedup vs baseline: 2.0273x; 1.5343x over previous
import functools

import jax
import jax.numpy as jnp
from jax import lax
from jax.experimental import pallas as pl
from jax.experimental.pallas import tpu as pltpu
from jax.experimental.pallas import tpu_sc as plsc

N = 10000
E = 320000
D = 128
DE = 16

NC = 2
NS = 16
NW = NC * NS
CHUNK = 128
CPT = 80
TOTAL_CHUNKS = CPT * NW
EPAD = TOTAL_CHUNKS * CHUNK
N_SC = 10240
RPT = N_SC // NS
RB = RPT // CHUNK
BE = 2048
BN = 512


def _gather_body(hv_hbm, ei_hbm, gs_hbm, sidx, rows, sem):
    cid = lax.axis_index("c")
    sid = lax.axis_index("s")
    wid = sid * NC + cid

    @pl.loop(0, CPT)
    def _edges(i):
        chunk = wid * CPT + i
        pltpu.sync_copy(ei_hbm.at[chunk, 0], sidx)
        pltpu.async_copy(hv_hbm.at[sidx], rows, sem).wait()
        pltpu.sync_copy(rows, gs_hbm.at[chunk])


_gather = functools.partial(
    pl.kernel,
    out_type=jax.ShapeDtypeStruct((TOTAL_CHUNKS, CHUNK, D), jnp.float32),
    mesh=plsc.VectorSubcoreMesh(core_axis_name="c", subcore_axis_name="s"),
    scratch_types=[
        pltpu.VMEM((CHUNK,), jnp.int32),
        pltpu.VMEM((CHUNK, D), jnp.float32),
        pltpu.SemaphoreType.DMA,
    ],
)(_gather_body)


def _deg_body(ei_hbm, out_hbm, didx, rows, acc, sem):
    cid = lax.axis_index("c")
    sid = lax.axis_index("s")
    wid = sid * NC + cid
    base = sid * RPT

    @pl.loop(0, CHUNK)
    def _zero(i):
        for j in range(D // 16):
            rows[i, pl.ds(j * 16, 16)] = jnp.zeros((16,), jnp.float32)

    for j in range(RB):
        pltpu.sync_copy(rows, acc.at[pl.ds(base + j * CHUNK, CHUNK)])

    @pl.loop(0, CHUNK)
    def _ones(i):
        for j in range(D // 16):
            rows[i, pl.ds(j * 16, 16)] = jnp.ones((16,), jnp.float32)

    plsc.subcore_barrier()

    @pl.loop(0, CPT)
    def _edges(i):
        chunk = wid * CPT + i
        pltpu.sync_copy(ei_hbm.at[chunk, 1], didx)
        pltpu.sync_copy(rows, acc.at[didx], add=True)

    plsc.subcore_barrier()
    for j in range(RB):
        pltpu.sync_copy(acc.at[pl.ds(base + j * CHUNK, CHUNK)], rows)
        pltpu.sync_copy(rows, out_hbm.at[cid, pl.ds(base + j * CHUNK, CHUNK)])


_deg = functools.partial(
    pl.kernel,
    out_type=jax.ShapeDtypeStruct((NC, N_SC, D), jnp.float32),
    mesh=plsc.VectorSubcoreMesh(core_axis_name="c", subcore_axis_name="s"),
    scratch_types=[
        pltpu.VMEM((CHUNK,), jnp.int32),
        pltpu.VMEM((CHUNK, D), jnp.float32),
        pltpu.VMEM_SHARED((N_SC, D), jnp.float32),
        pltpu.SemaphoreType.DMA,
    ],
)(_deg_body)


def _scatter_body(act_hbm, ei_hbm, out_hbm, didx, rows, acc, sem):
    cid = lax.axis_index("c")
    sid = lax.axis_index("s")
    wid = sid * NC + cid
    base = sid * RPT

    @pl.loop(0, CHUNK)
    def _zero(i):
        for j in range(D // 16):
            rows[i, pl.ds(j * 16, 16)] = jnp.zeros((16,), jnp.float32)

    for j in range(RB):
        pltpu.sync_copy(rows, acc.at[pl.ds(base + j * CHUNK, CHUNK)])
    plsc.subcore_barrier()

    @pl.loop(0, CPT)
    def _edges(i):
        chunk = wid * CPT + i
        pltpu.sync_copy(ei_hbm.at[chunk, 1], didx)
        pltpu.async_copy(act_hbm.at[chunk], rows, sem).wait()
        pltpu.sync_copy(rows, acc.at[didx], add=True)

    plsc.subcore_barrier()
    for j in range(RB):
        pltpu.sync_copy(acc.at[pl.ds(base + j * CHUNK, CHUNK)], rows)
        pltpu.sync_copy(rows, out_hbm.at[cid, pl.ds(base + j * CHUNK, CHUNK)])


_scatter = functools.partial(
    pl.kernel,
    out_type=jax.ShapeDtypeStruct((NC, N_SC, D), jnp.float32),
    mesh=plsc.VectorSubcoreMesh(core_axis_name="c", subcore_axis_name="s"),
    scratch_types=[
        pltpu.VMEM((CHUNK,), jnp.int32),
        pltpu.VMEM((CHUNK, D), jnp.float32),
        pltpu.VMEM_SHARED((N_SC, D), jnp.float32),
        pltpu.SemaphoreType.DMA,
    ],
)(_scatter_body)


def _act_body(gs, hee, WsT, WeT, bm_r, out):
    out[...] = (jnp.dot(gs[...], WsT[...], preferred_element_type=jnp.float32)
                + jnp.dot(hee[...], WeT[...], preferred_element_type=jnp.float32)
                + bm_r[...])


def _act(gs, hee, WsT, WeT, bm_r):
    const = lambda i: (0, 0)
    return pl.pallas_call(
        _act_body,
        grid=(EPAD // BE,),
        in_specs=[
            pl.BlockSpec((BE, D), lambda i: (i, 0)),
            pl.BlockSpec((BE, DE), lambda i: (i, 0)),
            pl.BlockSpec((D, D), const),
            pl.BlockSpec((DE, D), const),
            pl.BlockSpec((1, D), const),
        ],
        out_specs=pl.BlockSpec((BE, D), lambda i: (i, 0)),
        out_shape=jax.ShapeDtypeStruct((EPAD, D), jnp.float32),
    )(gs, hee, WsT, WeT, bm_r)


def _dense_body(ap, degp, hvp, WdT, WihT, bih_r, WhhT, bhh_r, out):
    hv = hvp[...]
    deg = degp[0][:, 0:1] + degp[1][:, 0:1]
    a = (ap[0] + ap[1]
         + deg * jnp.dot(hv, WdT[...], preferred_element_type=jnp.float32))
    gi = jnp.dot(a, WihT[...], preferred_element_type=jnp.float32) + bih_r[...]
    gh = jnp.dot(hv, WhhT[...], preferred_element_type=jnp.float32) + bhh_r[...]
    r = jax.nn.sigmoid(gi[:, :D] + gh[:, :D])
    z = jax.nn.sigmoid(gi[:, D:2 * D] + gh[:, D:2 * D])
    n = jnp.tanh(gi[:, 2 * D:] + r * gh[:, 2 * D:])
    out[...] = (1.0 - z) * n + z * hv


def _dense(ap, degp, hvp, WdT, WihT, bih_r, WhhT, bhh_r):
    const = lambda i: (0, 0)
    return pl.pallas_call(
        _dense_body,
        grid=(N_SC // BN,),
        in_specs=[
            pl.BlockSpec((NC, BN, D), lambda i: (0, i, 0)),
            pl.BlockSpec((NC, BN, D), lambda i: (0, i, 0)),
            pl.BlockSpec((BN, D), lambda i: (i, 0)),
            pl.BlockSpec((D, D), const),
            pl.BlockSpec((D, 3 * D), const),
            pl.BlockSpec((1, 3 * D), const),
            pl.BlockSpec((D, 3 * D), const),
            pl.BlockSpec((1, 3 * D), const),
        ],
        out_specs=pl.BlockSpec((BN, D), lambda i: (i, 0)),
        out_shape=jax.ShapeDtypeStruct((N_SC, D), jnp.float32),
    )(ap, degp, hvp, WdT, WihT, bih_r, WhhT, bhh_r)


def kernel(hv, edge_index, he,
           Wm0, bm0, Wih0, Whh0, bih0, bhh0,
           Wm1, bm1, Wih1, Whh1, bih1, bhh1):
    src = edge_index[0]
    dst = edge_index[1]
    pad_e = EPAD - E
    src_p = jnp.concatenate(
        [src, jnp.zeros((pad_e,), jnp.int32)]).reshape(TOTAL_CHUNKS, CHUNK)
    dst_p = jnp.concatenate(
        [dst, jnp.full((pad_e,), N, jnp.int32)]).reshape(TOTAL_CHUNKS, CHUNK)
    ei_p = jnp.stack([src_p, dst_p], axis=1)
    he_p = jnp.concatenate(
        [he, jnp.zeros((pad_e, DE), jnp.float32)])
    hv_p = jnp.pad(hv, ((0, N_SC - N), (0, 0)))

    degp = _deg(ei_p)

    for (Wm, bm, Wih, Whh, bih, bhh) in (
            (Wm0, bm0, Wih0, Whh0, bih0, bhh0),
            (Wm1, bm1, Wih1, Whh1, bih1, bhh1)):
        gs = _gather(hv_p, ei_p)
        act = _act(gs.reshape(EPAD, D), he_p,
                   Wm[:, :D].T, Wm[:, 2 * D:].T, bm.reshape(1, D))
        ap = _scatter(act.reshape(TOTAL_CHUNKS, CHUNK, D), ei_p)
        hv_p = _dense(ap, degp, hv_p, Wm[:, D:2 * D].T,
                      Wih.T, bih.reshape(1, 3 * D),
                      Whh.T, bhh.reshape(1, 3 * D))
    return hv_p[:N]
